# baseline (speedup 1.0000x reference)
import functools

import jax
import jax.numpy as jnp
from jax import lax
from jax.experimental import pallas as pl
from jax.experimental.pallas import tpu as pltpu
from jax.experimental.pallas import tpu_sc as plsc

B = 256
MARGIN = 0.2
EPS = 1e-8
BIG = 1e30

NC = 2
NS = 16
NW = NC * NS
L = 16
NCHUNK = B // L
APW = B // NW
PMAX = B


def _dist_kernel(x_ref, d_ref):
    x = x_ref[:, :]
    g = lax.dot_general(x, x, (((1,), (1,)), ((), ())),
                        preferred_element_type=jnp.float32)
    r = jnp.sum(x * x, axis=1)
    sq = r[:, None] + r[None, :] - 2.0 * g
    d_ref[:, :] = jnp.sqrt(jnp.maximum(sq, 1e-12))


_compute_dists = pl.pallas_call(
    _dist_kernel,
    out_shape=jax.ShapeDtypeStruct((B, B), jnp.float32),
)


@functools.partial(
    pl.kernel,
    out_type=(
        jax.ShapeDtypeStruct((B, PMAX), jnp.float32),
        jax.ShapeDtypeStruct((B,), jnp.int32),
    ),
    mesh=plsc.VectorSubcoreMesh(core_axis_name="c", subcore_axis_name="s"),
    scratch_types=[
        pltpu.VMEM((APW, B), jnp.float32),
        pltpu.VMEM((B,), jnp.int32),
        pltpu.VMEM((APW, PMAX), jnp.float32),
        pltpu.VMEM((APW,), jnp.int32),
    ],
    compiler_params=pltpu.CompilerParams(needs_layout_passes=False),
)
def _compact_sc(d_hbm, y_hbm, p_hbm, cnt_hbm, d_v, y_v, p_v, c_v):
    wid = lax.axis_index("s") * NC + lax.axis_index("c")
    base = wid * APW
    pltpu.sync_copy(y_hbm, y_v)
    pltpu.sync_copy(d_hbm.at[pl.ds(base, APW)], d_v)

    lane_iota = lax.iota(jnp.int32, L)
    zero_i = jnp.zeros((L,), jnp.int32)
    base_splat = zero_i + base
    lane0 = lane_iota == 0
    ya = [plsc.load_gather(y_v, [base_splat + i]) for i in range(APW)]

    pbases = [zero_i] * APW
    for j in range(NCHUNK):
        yj = y_v[pl.ds(j * L, L)]
        idxj = lane_iota + (j * L)
        for i in range(APW):
            dj = d_v[i, pl.ds(j * L, L)]
            posm = (yj == ya[i]) & (idxj != base_splat + i)
            dest = pbases[i] + plsc.cumsum(posm.astype(jnp.int32)) - 1
            dest = jnp.where(posm, dest, 0)
            plsc.store_scatter(p_v, [zero_i + i, dest], dj, mask=posm)
            pbases[i] = pbases[i] + plsc.all_reduce_population_count(posm)

    for i in range(APW):
        plsc.store_scatter(c_v, [zero_i + i], pbases[i], mask=lane0)

    pltpu.sync_copy(p_v, p_hbm.at[pl.ds(base, APW)])
    pltpu.sync_copy(c_v, cnt_hbm.at[pl.ds(base, APW)])


def _hinge_kernel(d_ref, p_ref, ycol_ref, yrow_ref, cntrow_ref, out_ref,
                  pt_ref):
    dmat = d_ref[:, :]
    same = ycol_ref[:, :] == yrow_ref[:, :]
    dneg = jnp.where(same, jnp.float32(BIG), dmat)

    cnt_row = cntrow_ref[:, :]
    slot_iota = lax.broadcasted_iota(jnp.int32, (B, PMAX), 1)
    cnt_col = jnp.transpose(cnt_row)
    p_clean = jnp.where(slot_iota < cnt_col, p_ref[:, :], jnp.float32(-BIG))

    a_i = lax.broadcasted_iota(jnp.int32, (B, B), 0)
    a_j = lax.broadcasted_iota(jnp.int32, (B, B), 1)
    eye = (a_i == a_j).astype(jnp.float32)
    pt_ref[:, :] = lax.dot_general(p_clean, eye, (((0,), (0,)), ((), ())),
                                   preferred_element_type=jnp.float32)

    hinge_c = jnp.float32(MARGIN - EPS)
    maxp = jnp.max(cnt_row)

    def body(t, acc):
        dp = pt_ref[pl.ds(t, 1), :]
        return acc + jnp.maximum(dp + hinge_c - dneg, 0.0)

    acc = lax.fori_loop(0, maxp, body, jnp.zeros((B, B), jnp.float32))

    npos = cnt_row.astype(jnp.float32)
    s = jnp.sum(acc) + jnp.float32(EPS) * jnp.sum(npos * (255.0 - npos))
    out_ref[0, 0] = s


_tc_hinge = pl.pallas_call(
    _hinge_kernel,
    out_specs=pl.BlockSpec(memory_space=pltpu.SMEM),
    out_shape=jax.ShapeDtypeStruct((1, 1), jnp.float32),
    scratch_shapes=[pltpu.VMEM((PMAX, B), jnp.float32)],
)


def kernel(x, y):
    d = _compute_dists(x)
    p, cnt = _compact_sc(d, y)
    s = _tc_hinge(d, p, y.reshape(B, 1), y.reshape(1, B), cnt.reshape(1, B))
    return s[0, 0]
